# Initial kernel scaffold; baseline (speedup 1.0000x reference)
import jax
import jax.numpy as jnp
from jax.experimental import pallas as pl


def kernel(x, W1e, b1e, W1a, b1a, W2e, b2e, W2a, b2a, W3e, b3e, W3a, b3a):
    raise NotImplementedError("write your pallas kernel here")


# speedup vs baseline: 2.2836x; 2.2836x over previous
import functools

import jax
import jax.numpy as jnp
from jax.experimental import pallas as pl
from jax.experimental.pallas import tpu as pltpu

_N = 7752
_K = 16
_RB = 128
_NBLK = 62
_NPAD = _RB * _NBLK
_CW = 256
_PREC = jax.lax.Precision.DEFAULT


def _knn_body(rows_ref, colsT_ref, out_ref, neg_ref):
    rows = rows_ref[...]
    colsT = colsT_ref[...]
    inner = jax.lax.dot_general(
        rows, colsT, (((1,), (0,)), ((), ())),
        preferred_element_type=jnp.float32, precision=_PREC)
    rows_sq = jnp.sum(rows * rows, axis=1, keepdims=True)
    cols_sq = jnp.sum(colsT * colsT, axis=0, keepdims=True)
    dist = (rows_sq + (-2.0) * inner) + cols_sq
    col_ids = jax.lax.broadcasted_iota(jnp.int32, (_RB, _NPAD), 1)
    neg_ref[...] = jnp.where(col_ids < _N, -dist, -jnp.inf)

    def chunk_step(c, carry):
        vals, ids = carry
        nchunk = neg_ref[:, pl.ds(c * _CW, _CW)]
        cids = c * _CW + jax.lax.broadcasted_iota(jnp.int32, (_RB, _CW), 1)
        V = jnp.concatenate([vals, nchunk], axis=1)
        I = jnp.concatenate([ids, cids], axis=1)
        new_v, new_i = [], []
        for _ in range(_K):
            m = jnp.max(V, axis=1, keepdims=True)
            sel = jnp.min(jnp.where(V == m, I, _NPAD), axis=1, keepdims=True)
            new_v.append(m)
            new_i.append(sel)
            V = jnp.where(I == sel, -jnp.inf, V)
        return (jnp.concatenate(new_v, axis=1), jnp.concatenate(new_i, axis=1))

    init = (jnp.full((_RB, _K), -jnp.inf, jnp.float32),
            jnp.full((_RB, _K), _NPAD, jnp.int32))
    _, ids = jax.lax.fori_loop(0, _NPAD // _CW, chunk_step, init)
    out_ref[...] = ids


def _knn(pts, ptsT):
    d = pts.shape[1]
    return pl.pallas_call(
        _knn_body,
        grid=(_NBLK,),
        in_specs=[
            pl.BlockSpec((_RB, d), lambda i: (i, 0)),
            pl.BlockSpec((d, _NPAD), lambda i: (0, 0)),
        ],
        out_specs=pl.BlockSpec((_RB, _K), lambda i: (i, 0)),
        out_shape=jax.ShapeDtypeStruct((_NPAD, _K), jnp.int32),
        scratch_shapes=[pltpu.VMEM((_RB, _NPAD), jnp.float32)],
    )(pts, ptsT)


def _edge_body(idx_ref, center_ref, table_ref, w_ref, b_ref, out_ref, xj_ref,
               *, c_in, o_out):
    center = center_ref[...]
    xi = jnp.broadcast_to(center[:, None, :], (_RB, _K, c_in)).reshape(
        _RB * _K, c_in)

    def n_body(n, _):
        def k_body(kk, _):
            j = idx_ref[0, n, kk]
            xj_ref[pl.ds(n * _K + kk, 1), :] = table_ref[pl.ds(j, 1), :]
            return 0
        return jax.lax.fori_loop(0, _K, k_body, 0)

    jax.lax.fori_loop(0, _RB, n_body, 0)

    cat = jnp.concatenate([xi, xj_ref[...] - xi], axis=1)
    z = jax.lax.dot_general(
        cat, w_ref[...], (((1,), (0,)), ((), ())),
        preferred_element_type=jnp.float32, precision=_PREC) + b_ref[...]
    ze = z[:, :o_out].reshape(_RB, _K, o_out)
    za = z[:, o_out:2 * o_out].reshape(_RB, _K, o_out)
    attn = jax.nn.softmax(za, axis=1)
    out_ref[...] = jnp.sum(ze * attn, axis=1)


def _edge(idx3d, table, wcat, bcat):
    c_in = table.shape[1]
    o_out = wcat.shape[1] // 2
    return pl.pallas_call(
        functools.partial(_edge_body, c_in=c_in, o_out=o_out),
        grid=(_NBLK,),
        in_specs=[
            pl.BlockSpec((1, _RB, _K), lambda i: (i, 0, 0),
                         memory_space=pltpu.SMEM),
            pl.BlockSpec((_RB, c_in), lambda i: (i, 0)),
            pl.BlockSpec((_NPAD, c_in), lambda i: (0, 0)),
            pl.BlockSpec((2 * c_in, 2 * o_out), lambda i: (0, 0)),
            pl.BlockSpec((1, 2 * o_out), lambda i: (0, 0)),
        ],
        out_specs=pl.BlockSpec((_RB, o_out), lambda i: (i, 0)),
        out_shape=jax.ShapeDtypeStruct((_NPAD, o_out), jnp.float32),
        scratch_shapes=[pltpu.VMEM((_RB * _K, c_in), jnp.float32)],
    )(idx3d, table, table, wcat, bcat)


def _stage(feats, pts, We, be, Wa, ba):
    idx = _knn(pts, pts.T)
    idx3d = idx.reshape(_NBLK, _RB, _K)
    wcat = jnp.concatenate([We.T, Wa.T], axis=1)
    bcat = jnp.concatenate([be, ba])[None, :]
    return _edge(idx3d, feats, wcat, bcat)


def kernel(x, W1e, b1e, W1a, b1a, W2e, b2e, W2a, b2a, W3e, b3e, W3a, b3a):
    xt = jnp.pad(x[0, :, :, 0].T, ((0, _NPAD - _N), (0, 0)))
    pts1 = xt[:, 0:3]
    feat1 = xt[:, 3:84]
    h1 = _stage(feat1, pts1, W1e, b1e, W1a, b1a)
    h2 = _stage(h1, h1, W2e, b2e, W2a, b2a)
    h3 = _stage(h2, h2, W3e, b3e, W3a, b3a)
    return jnp.transpose(h3[:_N])[None, :, :, None]
